# Initial kernel scaffold; baseline (speedup 1.0000x reference)
import jax
import jax.numpy as jnp
from jax.experimental import pallas as pl


def kernel(x, sample_index, mask_index, W, b):
    raise NotImplementedError("write your pallas kernel here")


# speedup vs baseline: 1.0133x; 1.0133x over previous
import jax
import jax.numpy as jnp
from jax.experimental import pallas as pl
from jax.experimental.pallas import tpu as pltpu

_B = 4
_NROWS = 197
_ED = 768
_NIDX = 99
_P = 50176
_BLK = 3584
_K = _P // _BLK
_SCALE = 127.0 / 255.0


def _body(idx_ref, mask_ref, x_ref, b_ref, w_ref, out_ref, acc_ref):
    k = pl.program_id(0)

    @pl.when(k == 0)
    def _init():
        acc_ref[...] = jnp.zeros_like(acc_ref)

    blk = w_ref[...]
    acc_ref[...] += jnp.sum(blk.reshape(_ED, _BLK // 128, 128), axis=1)

    @pl.when(k == _K - 1)
    def _finish():
        acc_t = jnp.transpose(acc_ref[...])
        m_row = jnp.sum(acc_t, axis=0, keepdims=True) * _SCALE + b_ref[...]

        idx = idx_ref[...]
        mask = mask_ref[...]

        j_col = jax.lax.broadcasted_iota(jnp.int32, (_NROWS, 1), 0)
        eq = idx == j_col
        pos = jax.lax.broadcasted_iota(jnp.int32, (_NROWS, 128), 1)
        lastpos = jnp.max(jnp.where(eq, pos, -1), axis=1, keepdims=True)
        is_mask = jnp.any(mask == j_col, axis=1, keepdims=True)
        sel = jnp.where(eq & (pos == lastpos) & jnp.logical_not(is_mask),
                        1.0, 0.0)
        mterm = is_mask.astype(jnp.float32) * m_row
        for bi in range(_B):
            out_ref[bi] = jax.lax.dot_general(
                sel, x_ref[bi], (((1,), (0,)), ((), ())),
                preferred_element_type=jnp.float32) + mterm


def kernel(x, sample_index, mask_index, W, b):
    idx_full = jnp.concatenate(
        [jnp.zeros((1,), sample_index.dtype), sample_index]).astype(jnp.int32)
    idx_p = jnp.full((1, 128), -1, jnp.int32).at[0, :_NIDX].set(idx_full)
    mask_p = jnp.full((1, 128), -1, jnp.int32).at[0, :98].set(
        mask_index.astype(jnp.int32))
    x_p = jnp.zeros((_B, 128, _ED), x.dtype).at[:, :_NIDX, :].set(x)
    b_row = b.reshape(1, _ED)

    return pl.pallas_call(
        _body,
        grid=(_K,),
        in_specs=[
            pl.BlockSpec((1, 128), lambda k: (0, 0)),
            pl.BlockSpec((1, 128), lambda k: (0, 0)),
            pl.BlockSpec((_B, 128, _ED), lambda k: (0, 0, 0)),
            pl.BlockSpec((1, _ED), lambda k: (0, 0)),
            pl.BlockSpec((_ED, _BLK), lambda k: (0, k)),
        ],
        out_specs=pl.BlockSpec((_B, _NROWS, _ED), lambda k: (0, 0, 0)),
        out_shape=jax.ShapeDtypeStruct((_B, _NROWS, _ED), jnp.float32),
        scratch_shapes=[pltpu.VMEM((_ED, 128), jnp.float32)],
    )(idx_p, mask_p, x_p, b_row, W)


# speedup vs baseline: 1.0610x; 1.0471x over previous
import jax
import jax.numpy as jnp
from jax.experimental import pallas as pl
from jax.experimental.pallas import tpu as pltpu

_B = 4
_NROWS = 197
_ED = 768
_NIDX = 99
_P = 50176
_BLK = 3584
_K = _P // _BLK
_SCALE = 127.0 / 255.0


_RB = 96


def _body(idx_ref, mask_ref, x_ref, b_ref, w_ref, out_ref, acc_ref):
    k = pl.program_id(0)

    blk = w_ref[...]
    acc_ref[pl.ds(k * _RB, _RB), :] = jnp.sum(
        blk.reshape(_RB, _P // 128, 128), axis=1)

    @pl.when(k == _ED // _RB - 1)
    def _finish():
        acc_t = jnp.transpose(acc_ref[...])
        m_row = jnp.sum(acc_t, axis=0, keepdims=True) * _SCALE + b_ref[...]

        idx = idx_ref[...]
        mask = mask_ref[...]

        j_col = jax.lax.broadcasted_iota(jnp.int32, (_NROWS, 1), 0)
        eq = idx == j_col
        pos = jax.lax.broadcasted_iota(jnp.int32, (_NROWS, 128), 1)
        lastpos = jnp.max(jnp.where(eq, pos, -1), axis=1, keepdims=True)
        is_mask = jnp.any(mask == j_col, axis=1, keepdims=True)
        sel = jnp.where(eq & (pos == lastpos) & jnp.logical_not(is_mask),
                        1.0, 0.0)
        mterm = is_mask.astype(jnp.float32) * m_row
        for bi in range(_B):
            out_ref[bi] = jax.lax.dot_general(
                sel, x_ref[bi], (((1,), (0,)), ((), ())),
                preferred_element_type=jnp.float32) + mterm


def kernel(x, sample_index, mask_index, W, b):
    idx_full = jnp.concatenate(
        [jnp.zeros((1,), sample_index.dtype), sample_index]).astype(jnp.int32)
    idx_p = jnp.full((1, 128), -1, jnp.int32).at[0, :_NIDX].set(idx_full)
    mask_p = jnp.full((1, 128), -1, jnp.int32).at[0, :98].set(
        mask_index.astype(jnp.int32))
    x_p = jnp.zeros((_B, 128, _ED), x.dtype).at[:, :_NIDX, :].set(x)
    b_row = b.reshape(1, _ED)

    return pl.pallas_call(
        _body,
        grid=(_ED // _RB,),
        in_specs=[
            pl.BlockSpec((1, 128), lambda k: (0, 0)),
            pl.BlockSpec((1, 128), lambda k: (0, 0)),
            pl.BlockSpec((_B, 128, _ED), lambda k: (0, 0, 0)),
            pl.BlockSpec((1, _ED), lambda k: (0, 0)),
            pl.BlockSpec((_RB, _P), lambda k: (k, 0)),
        ],
        out_specs=pl.BlockSpec((_B, _NROWS, _ED), lambda k: (0, 0, 0)),
        out_shape=jax.ShapeDtypeStruct((_B, _NROWS, _ED), jnp.float32),
        scratch_shapes=[pltpu.VMEM((_ED, 128), jnp.float32)],
    )(idx_p, mask_p, x_p, b_row, W)


# speedup vs baseline: 1.0936x; 1.0308x over previous
import jax
import jax.numpy as jnp
from jax.experimental import pallas as pl
from jax.experimental.pallas import tpu as pltpu

_B = 4
_NROWS = 197
_ED = 768
_NIDX = 99
_P = 50176
_BLK = 3584
_K = _P // _BLK
_SCALE = 127.0 / 255.0


_RB = 64


def _body(idx_ref, mask_ref, x_ref, b_ref, w_ref, out_ref, acc_ref):
    k = pl.program_id(0)

    blk = w_ref[...]
    acc_ref[pl.ds(k * _RB, _RB), :] = jnp.sum(
        blk.reshape(_RB, _P // 128, 128), axis=1)

    @pl.when(k == _ED // _RB - 1)
    def _finish():
        acc_t = jnp.transpose(acc_ref[...])
        m_row = jnp.sum(acc_t, axis=0, keepdims=True) * _SCALE + b_ref[...]

        idx = idx_ref[...]
        mask = mask_ref[...]

        j_col = jax.lax.broadcasted_iota(jnp.int32, (_NROWS, 1), 0)
        eq = idx == j_col
        pos = jax.lax.broadcasted_iota(jnp.int32, (_NROWS, 128), 1)
        lastpos = jnp.max(jnp.where(eq, pos, -1), axis=1, keepdims=True)
        is_mask = jnp.any(mask == j_col, axis=1, keepdims=True)
        sel = jnp.where(eq & (pos == lastpos) & jnp.logical_not(is_mask),
                        1.0, 0.0)
        mterm = is_mask.astype(jnp.float32) * m_row
        for bi in range(_B):
            out_ref[bi] = jax.lax.dot_general(
                sel, x_ref[bi], (((1,), (0,)), ((), ())),
                preferred_element_type=jnp.float32) + mterm


def kernel(x, sample_index, mask_index, W, b):
    idx_full = jnp.concatenate(
        [jnp.zeros((1,), sample_index.dtype), sample_index]).astype(jnp.int32)
    idx_p = jnp.full((1, 128), -1, jnp.int32).at[0, :_NIDX].set(idx_full)
    mask_p = jnp.full((1, 128), -1, jnp.int32).at[0, :98].set(
        mask_index.astype(jnp.int32))
    x_p = jnp.zeros((_B, 128, _ED), x.dtype).at[:, :_NIDX, :].set(x)
    b_row = b.reshape(1, _ED)

    return pl.pallas_call(
        _body,
        grid=(_ED // _RB,),
        in_specs=[
            pl.BlockSpec((1, 128), lambda k: (0, 0)),
            pl.BlockSpec((1, 128), lambda k: (0, 0)),
            pl.BlockSpec((_B, 128, _ED), lambda k: (0, 0, 0)),
            pl.BlockSpec((1, _ED), lambda k: (0, 0)),
            pl.BlockSpec((_RB, _P), lambda k: (k, 0)),
        ],
        out_specs=pl.BlockSpec((_B, _NROWS, _ED), lambda k: (0, 0, 0)),
        out_shape=jax.ShapeDtypeStruct((_B, _NROWS, _ED), jnp.float32),
        scratch_shapes=[pltpu.VMEM((_ED, 128), jnp.float32)],
    )(idx_p, mask_p, x_p, b_row, W)
